# baseline (speedup 1.0000x reference)
import functools

import jax
import jax.numpy as jnp
from jax import lax
from jax.experimental import pallas as pl
from jax.experimental.pallas import tpu as pltpu
from jax.experimental.pallas import tpu_sc as plsc

_NC = 2
_NS = 16
_NW = _NC * _NS

_B = 16384
_D = 64
_BPW = _B // _NW


def _gather_body(uhalf_hbm, ihalf_hbm, utab_hbm, itab_hbm, ue2_out, ie2_out,
                 idx_v, rows_v, sem):
    wid = lax.axis_index("s") * _NC + lax.axis_index("c")
    base = wid * _BPW
    pltpu.sync_copy(uhalf_hbm.at[pl.ds(base, _BPW)], idx_v)
    pltpu.async_copy(utab_hbm.at[idx_v], rows_v, sem).wait()
    pltpu.sync_copy(rows_v, ue2_out.at[pl.ds(base, _BPW)])
    pltpu.sync_copy(ihalf_hbm.at[pl.ds(base, _BPW)], idx_v)
    pltpu.async_copy(itab_hbm.at[idx_v], rows_v, sem).wait()
    pltpu.sync_copy(rows_v, ie2_out.at[pl.ds(base, _BPW)])


def _sc_gather():
    return pl.kernel(
        _gather_body,
        mesh=plsc.VectorSubcoreMesh(core_axis_name="c", subcore_axis_name="s"),
        out_type=[
            jax.ShapeDtypeStruct((_B, 2 * _D), jnp.float32),
            jax.ShapeDtypeStruct((_B, 2 * _D), jnp.float32),
        ],
        scratch_types=[
            pltpu.VMEM((_BPW,), jnp.int32),
            pltpu.VMEM((_BPW, 2 * _D), jnp.float32),
            pltpu.SemaphoreType.DMA,
        ],
    )


def _mlp_body(ue2, ie2, up, ip, w1a, w1b, b1, w2, b2, w3, b3, out):
    dot = functools.partial(jnp.dot, preferred_element_type=jnp.float32)
    ue = jnp.where(up[...] == 1, ue2[:, _D:], ue2[:, :_D])
    ie = jnp.where(ip[...] == 1, ie2[:, _D:], ie2[:, :_D])
    x = dot(ue, w1a[...]) + dot(ie, w1b[...]) + b1[...]
    x = jnp.maximum(x, 0.0)
    h = jnp.maximum(dot(x, w2[...]) + b2[...], 0.0)
    out[...] = dot(h, w3[...]) + b3[...]


def kernel(user, item, user_table, item_table, W1, b1, W2, b2, W3, b3):
    user = jnp.asarray(user, jnp.int32)
    item = jnp.asarray(item, jnp.int32)
    utab2 = user_table.reshape(500000, 2 * _D)
    itab2 = item_table.reshape(500000, 2 * _D)
    uhalf = lax.shift_right_logical(user, 1)
    ihalf = lax.shift_right_logical(item, 1)
    ue2, ie2 = _sc_gather()(uhalf, ihalf, utab2, itab2)

    up = lax.bitwise_and(user, 1).reshape(_B, 1)
    ip = lax.bitwise_and(item, 1).reshape(_B, 1)

    bs = 2048
    grid = (_B // bs,)
    full = lambda r, c: pl.BlockSpec((r, c), lambda i: (0, 0))
    out = pl.pallas_call(
        _mlp_body,
        grid=grid,
        in_specs=[
            pl.BlockSpec((bs, 2 * _D), lambda i: (i, 0)),
            pl.BlockSpec((bs, 2 * _D), lambda i: (i, 0)),
            pl.BlockSpec((bs, 1), lambda i: (i, 0)),
            pl.BlockSpec((bs, 1), lambda i: (i, 0)),
            full(_D, 64), full(_D, 64), full(1, 64),
            full(64, 32), full(1, 32),
            full(32, 1), full(1, 1),
        ],
        out_specs=pl.BlockSpec((bs, 1), lambda i: (i, 0)),
        out_shape=jax.ShapeDtypeStruct((_B, 1), jnp.float32),
    )(ue2, ie2, up, ip, W1[:_D], W1[_D:], b1.reshape(1, 64),
      W2, b2.reshape(1, 32), W3, b3.reshape(1, 1))
    return out


# speedup vs baseline: 1.5745x; 1.5745x over previous
import functools

import jax
import jax.numpy as jnp
from jax import lax
from jax.experimental import pallas as pl
from jax.experimental.pallas import tpu as pltpu
from jax.experimental.pallas import tpu_sc as plsc

_NC = 2
_NS = 16
_NW = _NC * _NS

_B = 16384
_D = 64
_BPW = _B // _NW
_CHUNK = 64
_NCHUNK = _BPW // _CHUNK


def _fire_chunk(tab_ref, idx_v, obuf_v, sem, j):
    copies = []
    for g in range(_CHUNK // 16):
        v = idx_v[pl.ds(j * _CHUNK + g * 16, 16)]
        for k in range(16):
            copies.append(pltpu.async_copy(
                tab_ref.at[pl.ds(v[k], 1), :],
                obuf_v.at[pl.ds(g * 16 + k, 1), :], sem))
    return copies


def _gather_body(user_hbm, item_hbm, utab_hbm, itab_hbm, ue_out, ie_out,
                 uidx_v, iidx_v, ubuf_v, ibuf_v, usem, isem):
    wid = lax.axis_index("s") * _NC + lax.axis_index("c")
    base = wid * _BPW
    pltpu.sync_copy(user_hbm.at[pl.ds(base, _BPW)], uidx_v)
    pltpu.sync_copy(item_hbm.at[pl.ds(base, _BPW)], iidx_v)

    def chunk_body(j, _):
        ucopies = _fire_chunk(utab_hbm, uidx_v, ubuf_v, usem, j)
        icopies = _fire_chunk(itab_hbm, iidx_v, ibuf_v, isem, j)
        for c in ucopies:
            c.wait()
        pltpu.sync_copy(ubuf_v,
                        ue_out.at[pl.ds(base + j * _CHUNK, _CHUNK)])
        for c in icopies:
            c.wait()
        pltpu.sync_copy(ibuf_v,
                        ie_out.at[pl.ds(base + j * _CHUNK, _CHUNK)])
        return ()

    lax.fori_loop(0, _NCHUNK, chunk_body, ())


def _sc_gather():
    return pl.kernel(
        _gather_body,
        mesh=plsc.VectorSubcoreMesh(core_axis_name="c", subcore_axis_name="s"),
        out_type=[
            jax.ShapeDtypeStruct((_B, _D), jnp.float32),
            jax.ShapeDtypeStruct((_B, _D), jnp.float32),
        ],
        scratch_types=[
            pltpu.VMEM((_BPW,), jnp.int32),
            pltpu.VMEM((_BPW,), jnp.int32),
            pltpu.VMEM((_CHUNK, _D), jnp.float32),
            pltpu.VMEM((_CHUNK, _D), jnp.float32),
            pltpu.SemaphoreType.DMA,
            pltpu.SemaphoreType.DMA,
        ],
    )


def _mlp_body(ue, ie, w1a, w1b, b1, w2, b2, w3, b3, out):
    dot = functools.partial(jnp.dot, preferred_element_type=jnp.float32)
    x = dot(ue[...], w1a[...]) + dot(ie[...], w1b[...]) + b1[...]
    x = jnp.maximum(x, 0.0)
    h = jnp.maximum(dot(x, w2[...]) + b2[...], 0.0)
    out[...] = dot(h, w3[...]) + b3[...]


def kernel(user, item, user_table, item_table, W1, b1, W2, b2, W3, b3):
    user = jnp.asarray(user, jnp.int32)
    item = jnp.asarray(item, jnp.int32)
    ue, ie = _sc_gather()(user, item, user_table, item_table)

    bs = 2048
    grid = (_B // bs,)
    full = lambda r, c: pl.BlockSpec((r, c), lambda i: (0, 0))
    out = pl.pallas_call(
        _mlp_body,
        grid=grid,
        in_specs=[
            pl.BlockSpec((bs, _D), lambda i: (i, 0)),
            pl.BlockSpec((bs, _D), lambda i: (i, 0)),
            full(_D, 64), full(_D, 64), full(1, 64),
            full(64, 32), full(1, 32),
            full(32, 1), full(1, 1),
        ],
        out_specs=pl.BlockSpec((bs, 1), lambda i: (i, 0)),
        out_shape=jax.ShapeDtypeStruct((_B, 1), jnp.float32),
    )(ue, ie, W1[:_D], W1[_D:], b1.reshape(1, 64),
      W2, b2.reshape(1, 32), W3, b3.reshape(1, 1))
    return out
